# baseline (speedup 1.0000x reference)
import jax
import jax.numpy as jnp
from jax import lax
from jax.experimental import pallas as pl
from jax.experimental.pallas import tpu as pltpu
from jax.experimental.pallas import tpu_sc as plsc

NC = 2
NS = 16
NW = NC * NS
L = 200
D = 64
B = 4096
V = 1000000
VP = V // 2
BPW = B // NW
NSUB = BPW // 16
NBUF = 4
LA = 2
NG = L // NBUF


def _emb_body(ids_hbm, table_hbm, pos_hbm, out_hbm,
              idx_v, idx2_v, pos_v, b0, b1, b2, b3, ob0, ob1,
              g0, g1, g2, g3, o0, o1):
    wid = lax.axis_index("s") * NC + lax.axis_index("c")
    row0 = wid * BPW
    pltpu.sync_copy(ids_hbm.at[:, pl.ds(row0, BPW)], idx_v)
    pltpu.sync_copy(pos_hbm, pos_v)

    bufs = (b0, b1, b2, b3)
    obufs = (ob0, ob1)
    gsems = (g0, g1, g2, g3)
    osems = (o0, o1)

    iota = lax.iota(jnp.int32, 16)

    def fill_idx2(l, b):
        for s in range(NSUB):
            v = idx_v[l, pl.ds(s * 16, 16)]
            idx2_v[b, pl.ds(s * 16, 16)] = v >> 1

    def gather(l, b):
        return pltpu.make_async_copy(
            table_hbm.at[idx2_v.at[b]], bufs[b], gsems[b])

    def outcopy(l, ob):
        return pltpu.make_async_copy(
            obufs[ob], out_hbm.at[l, :, pl.ds(row0, BPW)], osems[ob])

    fill_idx2(0, 0)
    gather(0, 0).start()
    fill_idx2(1, 1)
    gather(1, 1).start()

    @pl.loop(0, NG)
    def group(g):
        for b in range(NBUF):
            l = NBUF * g + b
            ob = b % 2
            buf = bufs[b]
            obuf = obufs[ob]
            gather(l, b).wait()
            if b >= 2:
                outcopy(l - 2, ob).wait()
            else:
                @pl.when(g >= 1)
                def _wait():
                    outcopy(l - 2, ob).wait()
            rowv = [iota + s * 16 for s in range(NSUB)]
            colb = []
            for s in range(NSUB):
                vi = idx_v[l, pl.ds(s * 16, 16)]
                colb.append((vi & 1) * 64)
            @pl.loop(0, D, unroll=4)
            def _dloop(d):
                pb = jnp.full((16,), l * D + d, jnp.int32)
                pv = plsc.load_gather(pos_v, [pb])
                for s in range(NSUB):
                    row = plsc.load_gather(buf, [rowv[s], colb[s] + d])
                    obuf[d, pl.ds(s * 16, 16)] = row + pv
            outcopy(l, ob).start()
            bn = (b + LA) % NBUF
            ln = l + LA
            if b < LA:
                fill_idx2(ln, bn)
                gather(ln, bn).start()
            else:
                @pl.when(g < NG - 1)
                def _go():
                    fill_idx2(ln, bn)
                    gather(ln, bn).start()

    outcopy(L - 2, 0).wait()
    outcopy(L - 1, 1).wait()


def kernel(input_ids, token_table, pos_table):
    mesh = plsc.VectorSubcoreMesh(core_axis_name="c", subcore_axis_name="s")
    f = pl.kernel(
        _emb_body,
        out_type=jax.ShapeDtypeStruct((L, D, B), jnp.float32),
        mesh=mesh,
        scratch_types=[
            pltpu.VMEM((L, BPW), jnp.int32),
            pltpu.VMEM((NBUF, BPW), jnp.int32),
            pltpu.VMEM((L * D,), jnp.float32),
            pltpu.VMEM((BPW, 128), jnp.float32),
            pltpu.VMEM((BPW, 128), jnp.float32),
            pltpu.VMEM((BPW, 128), jnp.float32),
            pltpu.VMEM((BPW, 128), jnp.float32),
            pltpu.VMEM((D, BPW), jnp.float32),
            pltpu.VMEM((D, BPW), jnp.float32),
        ] + [pltpu.SemaphoreType.DMA] * 6,
        compiler_params=pltpu.CompilerParams(
            use_tc_tiling_on_sc=True, needs_layout_passes=False),
    )
    ids_t = input_ids.astype(jnp.int32).T
    pos_flat = pos_table[:L].reshape(L * D)
    out = f(ids_t, token_table.reshape(VP, 128), pos_flat)
    return jnp.transpose(out, (2, 0, 1))


# speedup vs baseline: 1.5539x; 1.5539x over previous
import jax
import jax.numpy as jnp
from jax import lax
from jax.experimental import pallas as pl
from jax.experimental.pallas import tpu as pltpu
from jax.experimental.pallas import tpu_sc as plsc

NC = 2
NS = 16
NW = NC * NS
L = 200
D = 64
B = 4096
V = 1000000
BPW = B // NW
NBUF = 4
LA = 2
NG = L // NBUF


def _emb_body(ids_hbm, table_hbm, pos_hbm, out_hbm,
              idx_v, pos_v, b0, b1, b2, b3,
              g0, g1, g2, g3, o0, o1, o2, o3):
    wid = lax.axis_index("s") * NC + lax.axis_index("c")
    row0 = wid * BPW
    pltpu.sync_copy(ids_hbm.at[:, pl.ds(row0, BPW)], idx_v)
    pltpu.sync_copy(pos_hbm, pos_v)

    bufs = (b0, b1, b2, b3)
    gsems = (g0, g1, g2, g3)
    osems = (o0, o1, o2, o3)

    def gather(l, b):
        return pltpu.make_async_copy(table_hbm.at[idx_v.at[l]], bufs[b], gsems[b])

    def outcopy(l, b):
        return pltpu.make_async_copy(
            bufs[b], out_hbm.at[l, pl.ds(row0, BPW), :], osems[b])

    gather(0, 0).start()
    gather(1, 1).start()

    @pl.loop(0, NG)
    def group(g):
        for b in range(NBUF):
            l = NBUF * g + b
            buf = bufs[b]
            gather(l, b).wait()
            pv = [pos_v[pl.ds(l * D + q * 16, 16)] for q in range(D // 16)]
            for k in range(BPW):
                for q in range(D // 16):
                    plsc.addupdate(buf.at[k, pl.ds(q * 16, 16)], pv[q])
            outcopy(l, b).start()
            bn = (b + LA) % NBUF
            ln = l + LA
            if b < LA:
                @pl.when(g >= 1)
                def _wait():
                    outcopy(ln - NBUF, bn).wait()
                gather(ln, bn).start()
            else:
                outcopy(ln - NBUF, bn).wait()

                @pl.when(g < NG - 1)
                def _go():
                    gather(ln, bn).start()

    for b in range(LA, NBUF):
        outcopy(NBUF * (NG - 1) + b, b).wait()


def kernel(input_ids, token_table, pos_table):
    mesh = plsc.VectorSubcoreMesh(core_axis_name="c", subcore_axis_name="s")
    f = pl.kernel(
        _emb_body,
        out_type=jax.ShapeDtypeStruct((L, B, D), jnp.float32),
        mesh=mesh,
        scratch_types=[
            pltpu.VMEM((L, BPW), jnp.int32),
            pltpu.VMEM((L * D,), jnp.float32),
            pltpu.VMEM((BPW, D), jnp.float32),
            pltpu.VMEM((BPW, D), jnp.float32),
            pltpu.VMEM((BPW, D), jnp.float32),
            pltpu.VMEM((BPW, D), jnp.float32),
        ] + [pltpu.SemaphoreType.DMA] * 8,
        compiler_params=pltpu.CompilerParams(use_tc_tiling_on_sc=False),
    )
    ids_t = input_ids.astype(jnp.int32).T
    pos_flat = pos_table[:L].reshape(L * D)
    out = f(ids_t, token_table, pos_flat)
    return jnp.transpose(out, (1, 0, 2))
